# Initial kernel scaffold; baseline (speedup 1.0000x reference)
import jax
import jax.numpy as jnp
from jax.experimental import pallas as pl


def kernel(x, edge_index, W_in, b_in, ln_scale, ln_bias, ff1_W, ff1_b, ff2_W, ff2_b, out_ln_scale, out_ln_bias, W_out, b_out):
    raise NotImplementedError("write your pallas kernel here")


# speedup vs baseline: 4.3865x; 4.3865x over previous
import functools

import jax
import jax.numpy as jnp
from jax import lax
from jax.experimental import pallas as pl
from jax.experimental.pallas import tpu as pltpu
from jax.experimental.pallas import tpu_sc as plsc

N = 10000
D = 128
H = 128
HH = H // 2
E = 320000

NPAD = 10240
NSUB = 16
EW = 20480
EPAD = NSUB * EW
CHUNK = 128
NCHUNK = EW // CHUNK
ROWS_PER_TILE = NPAD // NSUB
RBLK = 1024
GRID = NPAD // RBLK

_HIGH = lax.Precision.HIGHEST


def _ln(h, scale, bias):
    mu = jnp.mean(h, axis=-1, keepdims=True)
    var = jnp.mean((h - mu) ** 2, axis=-1, keepdims=True)
    return (h - mu) * lax.rsqrt(var + 1e-5) * scale + bias


def _split_cols(z):
    return jnp.stack([z[:, :HH], z[:, HH:]])


def _tc_in_body(x_ref, wi_ref, bi_ref, s_ref, b_ref, h_ref, z_ref):
    h = jax.nn.gelu(
        jnp.dot(x_ref[...], wi_ref[...], preferred_element_type=jnp.float32,
                precision=_HIGH) + bi_ref[...])
    h_ref[...] = h
    z_ref[...] = _split_cols(_ln(h, s_ref[...], b_ref[...]))


def _agg_block(p_ref, d_ref):
    ps = jnp.concatenate([p_ref[0], p_ref[1]], axis=-1)
    dsum = d_ref[0] + d_ref[1]
    deg = jnp.maximum(dsum[:, 0:1], 1.0)
    return ps / deg


def _cat_z(z_ref):
    return jnp.concatenate([z_ref[0], z_ref[1]], axis=-1)


def _ffn(z, agg, h, w1a, w1b, b1, w2, b2):
    u = jax.nn.gelu(
        jnp.dot(z, w1a[...], preferred_element_type=jnp.float32, precision=_HIGH)
        + jnp.dot(agg, w1b[...], preferred_element_type=jnp.float32, precision=_HIGH)
        + b1[...])
    return h + jnp.dot(u, w2[...], preferred_element_type=jnp.float32,
                       precision=_HIGH) + b2[...]


def _tc_blk_body(h_ref, z_ref, p_ref, d_ref, w1a, w1b, b1, w2, b2, s_ref, b_ref,
                 h_out, z_out):
    agg = _agg_block(p_ref, d_ref)
    h2 = _ffn(_cat_z(z_ref), agg, h_ref[...], w1a, w1b, b1, w2, b2)
    h_out[...] = h2
    z_out[...] = _split_cols(_ln(h2, s_ref[...], b_ref[...]))


def _tc_fin_body(h_ref, z_ref, p_ref, d_ref, w1a, w1b, b1, w2, b2, so, bo,
                 wo, bco, out_ref):
    agg = _agg_block(p_ref, d_ref)
    h2 = _ffn(_cat_z(z_ref), agg, h_ref[...], w1a, w1b, b1, w2, b2)
    hn = _ln(h2, so[...], bo[...])
    out_ref[...] = jnp.sum(hn * wo[...], axis=-1, keepdims=True) + bco[0, 0]


def _row_spec(i):
    return (i, 0)


def _rep_spec(i):
    return (0, 0)


def _half_spec(i):
    return (0, i, 0)


_WSPEC = pl.BlockSpec((D, H), _rep_spec)
_VSPEC = pl.BlockSpec((1, H), _rep_spec)
_RSPEC = pl.BlockSpec((RBLK, H), _row_spec)
_ZSPEC = pl.BlockSpec((2, RBLK, HH), _half_spec)
_PSPEC = pl.BlockSpec((2, RBLK, HH), _half_spec)
_DSPEC = pl.BlockSpec((2, RBLK, 16), _half_spec)

_ZSHAPE = jax.ShapeDtypeStruct((2, NPAD, HH), jnp.float32)
_HSHAPE = jax.ShapeDtypeStruct((NPAD, H), jnp.float32)

_tc_in = pl.pallas_call(
    _tc_in_body,
    grid=(GRID,),
    in_specs=[_RSPEC, _WSPEC, _VSPEC, _VSPEC, _VSPEC],
    out_specs=[_RSPEC, _ZSPEC],
    out_shape=[_HSHAPE, _ZSHAPE],
)

_tc_blk = pl.pallas_call(
    _tc_blk_body,
    grid=(GRID,),
    in_specs=[_RSPEC, _ZSPEC, _PSPEC, _DSPEC,
              _WSPEC, _WSPEC, _VSPEC, _WSPEC, _VSPEC, _VSPEC, _VSPEC],
    out_specs=[_RSPEC, _ZSPEC],
    out_shape=[_HSHAPE, _ZSHAPE],
)

_tc_fin = pl.pallas_call(
    _tc_fin_body,
    grid=(GRID,),
    in_specs=[_RSPEC, _ZSPEC, _PSPEC, _DSPEC,
              _WSPEC, _WSPEC, _VSPEC, _WSPEC, _VSPEC, _VSPEC, _VSPEC,
              _VSPEC, pl.BlockSpec((1, 1), _rep_spec)],
    out_specs=[pl.BlockSpec((RBLK, 1), _row_spec)],
    out_shape=[jax.ShapeDtypeStruct((NPAD, 1), jnp.float32)],
)


def _sc_body(z_hbm, src_hbm, dst_hbm, agg_out, deg_out,
             src_v, dst_v, rows_v, ones_v, zero_v, zdeg_v,
             agg_s, deg_s, sem0, sem1):
    c = lax.axis_index("c")
    s = lax.axis_index("s")

    z16 = jnp.zeros((16,), jnp.float32)
    o16 = jnp.ones((16,), jnp.float32)

    def fill_row(r, carry):
        for j in range(HH // 16):
            zero_v[r, pl.ds(j * 16, 16)] = z16
        ones_v[r, pl.ds(0, 16)] = o16
        zdeg_v[r, pl.ds(0, 16)] = z16
        return carry

    lax.fori_loop(0, CHUNK, fill_row, 0)

    pltpu.sync_copy(src_hbm.at[s], src_v)
    pltpu.sync_copy(dst_hbm.at[s], dst_v)

    zc = z_hbm.at[c]
    pltpu.async_copy(zc.at[src_v.at[0]], rows_v.at[0], sem0)

    base = s * ROWS_PER_TILE

    def zero_blk(j, carry):
        off = base + j * CHUNK
        pltpu.sync_copy(zero_v, agg_s.at[pl.ds(off, CHUNK)])
        pltpu.sync_copy(zdeg_v, deg_s.at[pl.ds(off, CHUNK)])
        return carry

    lax.fori_loop(0, ROWS_PER_TILE // CHUNK, zero_blk, 0)

    plsc.subcore_barrier()

    def wait(buf_ref, sem):
        pltpu.make_async_copy(zc.at[src_v.at[0]], buf_ref, sem).wait()

    def body(i, carry):
        j0 = 2 * i
        wait(rows_v.at[0], sem0)
        pltpu.async_copy(zc.at[src_v.at[j0 + 1]], rows_v.at[1], sem1)
        pltpu.sync_copy(rows_v.at[0], agg_s.at[dst_v.at[j0]], add=True)

        @pl.when(c == 0)
        def _():
            pltpu.sync_copy(ones_v, deg_s.at[dst_v.at[j0]], add=True)

        wait(rows_v.at[1], sem1)

        @pl.when(i < NCHUNK // 2 - 1)
        def _():
            pltpu.async_copy(zc.at[src_v.at[j0 + 2]], rows_v.at[0], sem0)

        pltpu.sync_copy(rows_v.at[1], agg_s.at[dst_v.at[j0 + 1]], add=True)

        @pl.when(c == 1)
        def _():
            pltpu.sync_copy(ones_v, deg_s.at[dst_v.at[j0 + 1]], add=True)

        return carry

    lax.fori_loop(0, NCHUNK // 2, body, 0)

    plsc.subcore_barrier()

    pltpu.sync_copy(agg_s.at[pl.ds(base, ROWS_PER_TILE)],
                    agg_out.at[c, pl.ds(base, ROWS_PER_TILE)])
    pltpu.sync_copy(deg_s.at[pl.ds(base, ROWS_PER_TILE)],
                    deg_out.at[c, pl.ds(base, ROWS_PER_TILE)])


@functools.cache
def _get_sc_pass():
  return pl.kernel(
    _sc_body,
    out_type=[jax.ShapeDtypeStruct((2, NPAD, HH), jnp.float32),
              jax.ShapeDtypeStruct((2, NPAD, 16), jnp.float32)],
    mesh=plsc.VectorSubcoreMesh(core_axis_name="c", subcore_axis_name="s",
                                num_cores=2, num_subcores=NSUB),
    scratch_types=[
        pltpu.VMEM((NCHUNK, CHUNK), jnp.int32),
        pltpu.VMEM((NCHUNK, CHUNK), jnp.int32),
        pltpu.VMEM((2, CHUNK, HH), jnp.float32),
        pltpu.VMEM((CHUNK, 16), jnp.float32),
        pltpu.VMEM((CHUNK, HH), jnp.float32),
        pltpu.VMEM((CHUNK, 16), jnp.float32),
        pltpu.VMEM_SHARED((NPAD, HH), jnp.float32),
        pltpu.VMEM_SHARED((NPAD, 16), jnp.float32),
        pltpu.SemaphoreType.DMA,
        pltpu.SemaphoreType.DMA,
    ],
    compiler_params=pltpu.CompilerParams(use_tc_tiling_on_sc=False),
  )


def kernel(x, edge_index, W_in, b_in, ln_scale, ln_bias, ff1_W, ff1_b, ff2_W,
           ff2_b, out_ln_scale, out_ln_bias, W_out, b_out):
    xp = jnp.zeros((NPAD, D), jnp.float32).at[:N].set(x)
    src = edge_index[0]
    dst = edge_index[1]
    pad = EPAD - E
    src_p = jnp.concatenate([src, jnp.zeros((pad,), jnp.int32)])
    dst_p = jnp.concatenate(
        [dst, N + (jnp.arange(pad, dtype=jnp.int32) % (NPAD - N))])
    src3 = src_p.reshape(NSUB, NCHUNK, CHUNK)
    dst3 = dst_p.reshape(NSUB, NCHUNK, CHUNK)

    v = lambda a: a.reshape(1, H)
    sc_pass = _get_sc_pass()

    h, z = _tc_in(xp, W_in, v(b_in), v(ln_scale[0]), v(ln_bias[0]))
    aggp, degp = sc_pass(z, src3, dst3)
    h, z = _tc_blk(h, z, aggp, degp, ff1_W[0][:H], ff1_W[0][H:], v(ff1_b[0]),
                   ff2_W[0], v(ff2_b[0]), v(ln_scale[1]), v(ln_bias[1]))
    aggp, degp = sc_pass(z, src3, dst3)
    outp, = _tc_fin(h, z, aggp, degp, ff1_W[1][:H], ff1_W[1][H:], v(ff1_b[1]),
                    ff2_W[1], v(ff2_b[1]), v(out_ln_scale), v(out_ln_bias),
                    W_out.reshape(1, H), b_out.reshape(1, 1))
    return outp[:N, 0]
